# baseline (speedup 1.0000x reference)
import functools

import jax
import jax.numpy as jnp
from jax import lax
from jax.experimental import pallas as pl
from jax.experimental.pallas import tpu as pltpu
from jax.experimental.pallas import tpu_sc as plsc

N_NODES = 100000
N_PAD = 100096
HALF_N = 50000
D = 32
DH = 16
NNZ = 1600000
BATCH = 4096
NSUB = 16
NCORE = 2
CH = 400
NCHUNK = NNZ // NSUB // CH
STRIPE = N_PAD // NSUB
BSUB = BATCH // NSUB
NZCOPY = STRIPE // CH
ZREM = STRIPE - NZCOPY * CH

_mesh = plsc.VectorSubcoreMesh(core_axis_name="c", subcore_axis_name="s")


@functools.partial(
    pl.kernel,
    out_type=(
        jax.ShapeDtypeStruct((NCORE, BATCH), jnp.float32),
        jax.ShapeDtypeStruct((NCORE, BATCH), jnp.float32),
        jax.ShapeDtypeStruct((NCORE, N_PAD, DH), jnp.float32),
        jax.ShapeDtypeStruct((NCORE, N_PAD, DH), jnp.float32),
    ),
    mesh=_mesh,
    compiler_params=pltpu.CompilerParams(use_tc_tiling_on_sc=False),
    scratch_types=[
        pltpu.VMEM((3, CH), jnp.int32),
        pltpu.VMEM((3, CH), jnp.int32),
        pltpu.VMEM((CH, DH), jnp.float32),
        pltpu.VMEM((CH, DH), jnp.float32),
        pltpu.VMEM((BSUB,), jnp.int32),
        pltpu.VMEM((BSUB, DH), jnp.float32),
        pltpu.VMEM((BSUB, DH), jnp.float32),
        pltpu.VMEM((BSUB,), jnp.float32),
        pltpu.VMEM((BSUB,), jnp.float32),
        pltpu.VMEM_SHARED((N_PAD, DH), jnp.float32),
        pltpu.SemaphoreType.DMA,
        pltpu.SemaphoreType.DMA,
        pltpu.SemaphoreType.DMA,
        pltpu.SemaphoreType.DMA,
        pltpu.SemaphoreType.DMA,
    ],
)
def _lightgcn(x_hbm, edges_hbm, user_hbm, posi_hbm, negi_hbm,
              pos_out, neg_out, e1_hbm, e2_hbm,
              idx0, idx1, gath0, gath1,
              idxb, sum_a, sum_b, pos_buf, neg_buf,
              acc, sem_i0, sem_i1, sem_g0, sem_g1, sem_s):
    h = lax.axis_index("c")
    sid = lax.axis_index("s")
    base = sid * STRIPE
    nnz0 = sid * (NNZ // NSUB)
    lanes = lax.iota(jnp.int32, 16)

    idx = (idx0, idx1)
    gath = (gath0, gath1)
    sem_i = (sem_i0, sem_i1)
    sem_g = (sem_g0, sem_g1)

    def _take(v, idx):
        return lax.gather(
            v, idx.reshape(16, 1),
            lax.GatherDimensionNumbers(
                offset_dims=(), collapsed_slice_dims=(0,),
                start_index_map=(0,)),
            (1,), mode=lax.GatherScatterMode.PROMISE_IN_BOUNDS)

    def _hsum(v):
        for sh in (8, 4, 2, 1):
            v = v + _take(v, lanes ^ sh)
        return v

    def fire_idx(k, b):
        off = nnz0 + k * CH
        pltpu.async_copy(edges_hbm.at[:, pl.ds(off, CH)], idx[b], sem_i[b])

    def wait_idx(b):
        pltpu.make_async_copy(edges_hbm.at[:, pl.ds(0, CH)], idx[b],
                              sem_i[b]).wait()

    def start_gather(b, src):
        pltpu.async_copy(src.at[h].at[idx[b].at[1]], gath[b], sem_g[b])

    def process(b, src):
        pltpu.make_async_copy(src.at[h].at[idx[b].at[1]], gath[b],
                              sem_g[b]).wait()
        vbuf = idx[b]
        g = gath[b]

        @plsc.parallel_loop(0, CH // 16)
        def _scale(gi):
            vv = lax.bitcast_convert_type(vbuf[2, pl.ds(gi * 16, 16)],
                                          jnp.float32)
            for k in range(16):
                r = gi * 16 + k
                g[r] = g[r] * _take(vv, jnp.full((16,), k, jnp.int32))

        pltpu.sync_copy(g, acc.at[idx[b].at[0]], add=True)

    def spmm(src):
        fire_idx(0, 0)

        def outer(kk, carry):
            k0 = kk * 2
            wait_idx(0)
            start_gather(0, src)

            @pl.when(kk > 0)
            def _p1():
                process(1, src)

            fire_idx(k0 + 1, 1)
            wait_idx(1)
            start_gather(1, src)
            process(0, src)

            @pl.when(kk < NCHUNK // 2 - 1)
            def _f0():
                fire_idx(k0 + 2, 0)

            return carry

        lax.fori_loop(0, NCHUNK // 2, outer, None)
        process(1, src)

    def zero_acc():
        @plsc.parallel_loop(0, CH, unroll=4)
        def _z(r):
            gath0[r] = jnp.zeros((DH,), jnp.float32)

        for j in range(NZCOPY):
            pltpu.async_copy(gath0, acc.at[pl.ds(base + j * CH, CH)], sem_s)
        pltpu.async_copy(gath0.at[pl.ds(0, ZREM)],
                         acc.at[pl.ds(base + NZCOPY * CH, ZREM)], sem_s)
        for j in range(NZCOPY):
            pltpu.make_async_copy(gath0, acc.at[pl.ds(base + j * CH, CH)],
                                  sem_s).wait()
        pltpu.make_async_copy(gath0.at[pl.ds(0, ZREM)],
                              acc.at[pl.ds(base + NZCOPY * CH, ZREM)],
                              sem_s).wait()

    def dump(dst):
        pltpu.sync_copy(acc.at[pl.ds(base, STRIPE)],
                        dst.at[h].at[pl.ds(base, STRIPE)])

    zero_acc()
    plsc.subcore_barrier()
    spmm(x_hbm)
    plsc.subcore_barrier()
    dump(e1_hbm)
    zero_acc()
    plsc.subcore_barrier()
    spmm(e1_hbm)
    plsc.subcore_barrier()
    dump(e2_hbm)
    plsc.subcore_barrier()

    b0 = sid * BSUB

    def gather_sum(ids_hbm, dst):
        pltpu.sync_copy(ids_hbm.at[pl.ds(b0, BSUB)], idxb)
        pltpu.async_copy(x_hbm.at[h].at[idxb], dst, sem_g0)
        pltpu.async_copy(e1_hbm.at[h].at[idxb], gath0.at[pl.ds(0, BSUB)], sem_g0)
        pltpu.async_copy(e2_hbm.at[h].at[idxb], gath1.at[pl.ds(0, BSUB)], sem_g0)
        pltpu.make_async_copy(x_hbm.at[h].at[idxb], dst, sem_g0).wait()
        pltpu.make_async_copy(e1_hbm.at[h].at[idxb], gath0.at[pl.ds(0, BSUB)],
                              sem_g0).wait()
        pltpu.make_async_copy(e2_hbm.at[h].at[idxb], gath1.at[pl.ds(0, BSUB)],
                              sem_g0).wait()

        @plsc.parallel_loop(0, BSUB, unroll=4)
        def _acc(r):
            dst[r] = dst[r] + gath0[r] + gath1[r]

    def dots(out_buf):
        @plsc.parallel_loop(0, BSUB // 16)
        def _dot(gi):
            s_acc = jnp.zeros((16,), jnp.float32)
            for k in range(16):
                bb = gi * 16 + k
                s = _hsum(sum_a[bb] * sum_b[bb])
                s_acc = jnp.where(lanes == k, s, s_acc)
            out_buf[pl.ds(gi * 16, 16)] = s_acc

    gather_sum(user_hbm, sum_a)
    gather_sum(posi_hbm, sum_b)
    dots(pos_buf)
    gather_sum(negi_hbm, sum_b)
    dots(neg_buf)

    pltpu.sync_copy(pos_buf, pos_out.at[h].at[pl.ds(b0, BSUB)])
    pltpu.sync_copy(neg_buf, neg_out.at[h].at[pl.ds(b0, BSUB)])


def kernel(user_emb_w, item_emb_w, snm_vals, snm_rows, snm_cols, user, pos_item, neg_item):
    e0 = jnp.concatenate([user_emb_w, item_emb_w,
                          jnp.zeros((N_PAD - N_NODES, D), jnp.float32)], axis=0)
    x2 = jnp.stack([e0[:, :DH], e0[:, DH:]])
    edges = jnp.stack([snm_rows, snm_cols,
                       lax.bitcast_convert_type(snm_vals, jnp.int32)])
    pos_p, neg_p, _e1, _e2 = _lightgcn(
        x2, edges, user, pos_item + HALF_N, neg_item + HALF_N)
    pos = ((pos_p[0] + pos_p[1]) * (1.0 / 9.0)).reshape(BATCH, 1)
    neg = ((neg_p[0] + neg_p[1]) * (1.0 / 9.0)).reshape(BATCH, 1)
    return (pos, neg)


# speedup vs baseline: 1.3323x; 1.3323x over previous
import functools

import jax
import jax.numpy as jnp
from jax import lax
from jax.experimental import pallas as pl
from jax.experimental.pallas import tpu as pltpu
from jax.experimental.pallas import tpu_sc as plsc

N_NODES = 100000
N_PAD = 100096
HALF_N = 50000
D = 32
DH = 16
NNZ = 1600000
BATCH = 4096
NSUB = 16
NCORE = 2
CH = 400
NCHUNK = NNZ // NSUB // CH
STRIPE = N_PAD // NSUB
BSUB = BATCH // NSUB
NZCOPY = STRIPE // CH
ZREM = STRIPE - NZCOPY * CH

_mesh = plsc.VectorSubcoreMesh(core_axis_name="c", subcore_axis_name="s")


@functools.partial(
    pl.kernel,
    out_type=(
        jax.ShapeDtypeStruct((NCORE, BATCH), jnp.float32),
        jax.ShapeDtypeStruct((NCORE, BATCH), jnp.float32),
        jax.ShapeDtypeStruct((NCORE, N_PAD, DH), jnp.float32),
        jax.ShapeDtypeStruct((NCORE, N_PAD, DH), jnp.float32),
    ),
    mesh=_mesh,
    compiler_params=pltpu.CompilerParams(use_tc_tiling_on_sc=False),
    scratch_types=[
        pltpu.VMEM((3, CH), jnp.int32),
        pltpu.VMEM((3, CH), jnp.int32),
        pltpu.VMEM((CH, DH), jnp.float32),
        pltpu.VMEM((CH, DH), jnp.float32),
        pltpu.VMEM((BSUB,), jnp.int32),
        pltpu.VMEM((BSUB, DH), jnp.float32),
        pltpu.VMEM((BSUB, DH), jnp.float32),
        pltpu.VMEM((BSUB,), jnp.float32),
        pltpu.VMEM((BSUB,), jnp.float32),
        pltpu.VMEM_SHARED((N_PAD, DH), jnp.float32),
        pltpu.SemaphoreType.DMA,
        pltpu.SemaphoreType.DMA,
        pltpu.SemaphoreType.DMA,
        pltpu.SemaphoreType.DMA,
        pltpu.SemaphoreType.DMA,
    ],
)
def _lightgcn(x_hbm, edges_hbm, user_hbm, posi_hbm, negi_hbm,
              pos_out, neg_out, e1_hbm, e2_hbm,
              idx0, idx1, gath0, gath1,
              idxb, sum_a, sum_b, pos_buf, neg_buf,
              acc, sem_i0, sem_i1, sem_g0, sem_g1, sem_s):
    h = lax.axis_index("c")
    sid = lax.axis_index("s")
    base = sid * STRIPE
    nnz0 = sid * (NNZ // NSUB)
    lanes = lax.iota(jnp.int32, 16)

    idx = (idx0, idx1)
    gath = (gath0, gath1)
    sem_i = (sem_i0, sem_i1)
    sem_g = (sem_g0, sem_g1)

    def _take(v, idx):
        return lax.gather(
            v, idx.reshape(16, 1),
            lax.GatherDimensionNumbers(
                offset_dims=(), collapsed_slice_dims=(0,),
                start_index_map=(0,)),
            (1,), mode=lax.GatherScatterMode.PROMISE_IN_BOUNDS)

    def _hsum(v):
        for sh in (8, 4, 2, 1):
            v = v + _take(v, lanes ^ sh)
        return v

    def fire_idx(k, b):
        g = sid * NCHUNK + k
        pltpu.async_copy(edges_hbm.at[g], idx[b], sem_i[b])

    def wait_idx(b):
        pltpu.make_async_copy(edges_hbm.at[0], idx[b], sem_i[b]).wait()

    def start_gather(b, src):
        pltpu.async_copy(src.at[h].at[idx[b].at[1]], gath[b], sem_g[b])

    def process(b, src):
        pltpu.make_async_copy(src.at[h].at[idx[b].at[1]], gath[b],
                              sem_g[b]).wait()
        vbuf = idx[b]
        g = gath[b]

        @plsc.parallel_loop(0, CH // 16)
        def _scale(gi):
            vv = lax.bitcast_convert_type(vbuf[2, pl.ds(gi * 16, 16)],
                                          jnp.float32)
            for k in range(16):
                r = gi * 16 + k
                g[r] = g[r] * _take(vv, jnp.full((16,), k, jnp.int32))

        pltpu.sync_copy(g, acc.at[idx[b].at[0]], add=True)

    def spmm(src):
        fire_idx(0, 0)

        def outer(kk, carry):
            k0 = kk * 2
            wait_idx(0)
            start_gather(0, src)

            @pl.when(kk > 0)
            def _p1():
                process(1, src)

            fire_idx(k0 + 1, 1)
            wait_idx(1)
            start_gather(1, src)
            process(0, src)

            @pl.when(kk < NCHUNK // 2 - 1)
            def _f0():
                fire_idx(k0 + 2, 0)

            return carry

        lax.fori_loop(0, NCHUNK // 2, outer, None)
        process(1, src)

    def zero_acc():
        @plsc.parallel_loop(0, CH, unroll=4)
        def _z(r):
            gath0[r] = jnp.zeros((DH,), jnp.float32)

        for j in range(NZCOPY):
            pltpu.async_copy(gath0, acc.at[pl.ds(base + j * CH, CH)], sem_s)
        pltpu.async_copy(gath0.at[pl.ds(0, ZREM)],
                         acc.at[pl.ds(base + NZCOPY * CH, ZREM)], sem_s)
        for j in range(NZCOPY):
            pltpu.make_async_copy(gath0, acc.at[pl.ds(base + j * CH, CH)],
                                  sem_s).wait()
        pltpu.make_async_copy(gath0.at[pl.ds(0, ZREM)],
                              acc.at[pl.ds(base + NZCOPY * CH, ZREM)],
                              sem_s).wait()

    def dump(dst):
        pltpu.sync_copy(acc.at[pl.ds(base, STRIPE)],
                        dst.at[h].at[pl.ds(base, STRIPE)])

    zero_acc()
    plsc.subcore_barrier()
    spmm(x_hbm)
    plsc.subcore_barrier()
    dump(e1_hbm)
    zero_acc()
    plsc.subcore_barrier()
    spmm(e1_hbm)
    plsc.subcore_barrier()
    dump(e2_hbm)
    plsc.subcore_barrier()

    b0 = sid * BSUB

    def gather_sum(ids_hbm, dst):
        pltpu.sync_copy(ids_hbm.at[pl.ds(b0, BSUB)], idxb)
        pltpu.async_copy(x_hbm.at[h].at[idxb], dst, sem_g0)
        pltpu.async_copy(e1_hbm.at[h].at[idxb], gath0.at[pl.ds(0, BSUB)], sem_g0)
        pltpu.async_copy(e2_hbm.at[h].at[idxb], gath1.at[pl.ds(0, BSUB)], sem_g0)
        pltpu.make_async_copy(x_hbm.at[h].at[idxb], dst, sem_g0).wait()
        pltpu.make_async_copy(e1_hbm.at[h].at[idxb], gath0.at[pl.ds(0, BSUB)],
                              sem_g0).wait()
        pltpu.make_async_copy(e2_hbm.at[h].at[idxb], gath1.at[pl.ds(0, BSUB)],
                              sem_g0).wait()

        @plsc.parallel_loop(0, BSUB, unroll=4)
        def _acc(r):
            dst[r] = dst[r] + gath0[r] + gath1[r]

    def dots(out_buf):
        @plsc.parallel_loop(0, BSUB // 16)
        def _dot(gi):
            s_acc = jnp.zeros((16,), jnp.float32)
            for k in range(16):
                bb = gi * 16 + k
                s = _hsum(sum_a[bb] * sum_b[bb])
                s_acc = jnp.where(lanes == k, s, s_acc)
            out_buf[pl.ds(gi * 16, 16)] = s_acc

    gather_sum(user_hbm, sum_a)
    gather_sum(posi_hbm, sum_b)
    dots(pos_buf)
    gather_sum(negi_hbm, sum_b)
    dots(neg_buf)

    pltpu.sync_copy(pos_buf, pos_out.at[h].at[pl.ds(b0, BSUB)])
    pltpu.sync_copy(neg_buf, neg_out.at[h].at[pl.ds(b0, BSUB)])


def kernel(user_emb_w, item_emb_w, snm_vals, snm_rows, snm_cols, user, pos_item, neg_item):
    e0 = jnp.concatenate([user_emb_w, item_emb_w,
                          jnp.zeros((N_PAD - N_NODES, D), jnp.float32)], axis=0)
    x2 = jnp.stack([e0[:, :DH], e0[:, DH:]])
    edges = jnp.stack([snm_rows.reshape(-1, CH), snm_cols.reshape(-1, CH),
                       lax.bitcast_convert_type(snm_vals, jnp.int32)
                          .reshape(-1, CH)], axis=1)
    pos_p, neg_p, _e1, _e2 = _lightgcn(
        x2, edges, user, pos_item + HALF_N, neg_item + HALF_N)
    pos = ((pos_p[0] + pos_p[1]) * (1.0 / 9.0)).reshape(BATCH, 1)
    neg = ((neg_p[0] + neg_p[1]) * (1.0 / 9.0)).reshape(BATCH, 1)
    return (pos, neg)


# speedup vs baseline: 1.5846x; 1.1893x over previous
import functools

import jax
import jax.numpy as jnp
from jax import lax
from jax.experimental import pallas as pl
from jax.experimental.pallas import tpu as pltpu
from jax.experimental.pallas import tpu_sc as plsc

N_NODES = 100000
N_PAD = 100096
HALF_N = 50000
D = 32
DH = 16
NNZ = 1600000
BATCH = 4096
NSUB = 16
NCORE = 2
CH = 400
NCHUNK = NNZ // NSUB // CH
STRIPE = N_PAD // NSUB
BSUB = BATCH // NSUB
NZCOPY = STRIPE // CH
ZREM = STRIPE - NZCOPY * CH

_mesh = plsc.VectorSubcoreMesh(core_axis_name="c", subcore_axis_name="s")


@functools.partial(
    pl.kernel,
    out_type=(
        jax.ShapeDtypeStruct((NCORE, BATCH), jnp.float32),
        jax.ShapeDtypeStruct((NCORE, BATCH), jnp.float32),
        jax.ShapeDtypeStruct((NCORE, N_PAD, DH), jnp.float32),
        jax.ShapeDtypeStruct((NCORE, N_PAD, DH), jnp.float32),
    ),
    mesh=_mesh,
    compiler_params=pltpu.CompilerParams(use_tc_tiling_on_sc=False),
    scratch_types=[
        pltpu.VMEM((CH,), jnp.int32),
        pltpu.VMEM((CH,), jnp.int32),
        pltpu.VMEM((CH,), jnp.int32),
        pltpu.VMEM((CH,), jnp.int32),
        pltpu.VMEM((CH,), jnp.float32),
        pltpu.VMEM((CH,), jnp.float32),
        pltpu.VMEM((CH, DH), jnp.float32),
        pltpu.VMEM((CH, DH), jnp.float32),
        pltpu.VMEM((BSUB,), jnp.int32),
        pltpu.VMEM((BSUB, DH), jnp.float32),
        pltpu.VMEM((BSUB, DH), jnp.float32),
        pltpu.VMEM((BSUB,), jnp.float32),
        pltpu.VMEM((BSUB,), jnp.float32),
        pltpu.VMEM_SHARED((N_PAD, DH), jnp.float32),
        pltpu.SemaphoreType.DMA,
        pltpu.SemaphoreType.DMA,
        pltpu.SemaphoreType.DMA,
        pltpu.SemaphoreType.DMA,
        pltpu.SemaphoreType.DMA,
    ],
)
def _lightgcn(x_hbm, rows_hbm, cols_hbm, vals_hbm, user_hbm, posi_hbm, negi_hbm,
              pos_out, neg_out, e1_hbm, e2_hbm,
              rows0, rows1, cols0, cols1, vals0, vals1, gath0, gath1,
              idxb, sum_a, sum_b, pos_buf, neg_buf,
              acc, sem_i0, sem_i1, sem_g0, sem_g1, sem_s):
    h = lax.axis_index("c")
    sid = lax.axis_index("s")
    base = sid * STRIPE
    nnz0 = sid * (NNZ // NSUB)
    lanes = lax.iota(jnp.int32, 16)

    rows = (rows0, rows1)
    cols = (cols0, cols1)
    vals = (vals0, vals1)
    gath = (gath0, gath1)
    sem_i = (sem_i0, sem_i1)
    sem_g = (sem_g0, sem_g1)

    def _take(v, idx):
        return lax.gather(
            v, idx.reshape(16, 1),
            lax.GatherDimensionNumbers(
                offset_dims=(), collapsed_slice_dims=(0,),
                start_index_map=(0,)),
            (1,), mode=lax.GatherScatterMode.PROMISE_IN_BOUNDS)

    def _hsum(v):
        for sh in (8, 4, 2, 1):
            v = v + _take(v, lanes ^ sh)
        return v

    def fire_idx(k, b):
        off = nnz0 + k * CH
        pltpu.async_copy(rows_hbm.at[pl.ds(off, CH)], rows[b], sem_i[b])
        pltpu.async_copy(cols_hbm.at[pl.ds(off, CH)], cols[b], sem_i[b])
        pltpu.async_copy(vals_hbm.at[pl.ds(off, CH)], vals[b], sem_i[b])

    def wait_idx(b):
        pltpu.make_async_copy(rows_hbm.at[pl.ds(0, CH)], rows[b], sem_i[b]).wait()
        pltpu.make_async_copy(cols_hbm.at[pl.ds(0, CH)], cols[b], sem_i[b]).wait()
        pltpu.make_async_copy(vals_hbm.at[pl.ds(0, CH)], vals[b], sem_i[b]).wait()

    def start_gather(b, src):
        pltpu.async_copy(src.at[h].at[cols[b]], gath[b], sem_g[b])

    def process(b, src):
        pltpu.make_async_copy(src.at[h].at[cols[b]], gath[b], sem_g[b]).wait()
        vbuf = vals[b]
        g = gath[b]

        @plsc.parallel_loop(0, CH // 16)
        def _scale(gi):
            vv = vbuf[pl.ds(gi * 16, 16)]
            for k in range(16):
                r = gi * 16 + k
                g[r] = g[r] * _take(vv, jnp.full((16,), k, jnp.int32))

        pltpu.sync_copy(g, acc.at[rows[b]], add=True)

    def spmm(src):
        fire_idx(0, 0)

        def outer(kk, carry):
            k0 = kk * 2
            wait_idx(0)
            start_gather(0, src)

            @pl.when(kk > 0)
            def _p1():
                process(1, src)

            fire_idx(k0 + 1, 1)
            wait_idx(1)
            start_gather(1, src)
            process(0, src)

            @pl.when(kk < NCHUNK // 2 - 1)
            def _f0():
                fire_idx(k0 + 2, 0)

            return carry

        lax.fori_loop(0, NCHUNK // 2, outer, None)
        process(1, src)

    def zero_acc():
        @plsc.parallel_loop(0, CH, unroll=4)
        def _z(r):
            gath0[r] = jnp.zeros((DH,), jnp.float32)

        for j in range(NZCOPY):
            pltpu.async_copy(gath0, acc.at[pl.ds(base + j * CH, CH)], sem_s)
        pltpu.async_copy(gath0.at[pl.ds(0, ZREM)],
                         acc.at[pl.ds(base + NZCOPY * CH, ZREM)], sem_s)
        for j in range(NZCOPY):
            pltpu.make_async_copy(gath0, acc.at[pl.ds(base + j * CH, CH)],
                                  sem_s).wait()
        pltpu.make_async_copy(gath0.at[pl.ds(0, ZREM)],
                              acc.at[pl.ds(base + NZCOPY * CH, ZREM)],
                              sem_s).wait()

    def dump(dst):
        pltpu.sync_copy(acc.at[pl.ds(base, STRIPE)],
                        dst.at[h].at[pl.ds(base, STRIPE)])

    zero_acc()
    plsc.subcore_barrier()
    spmm(x_hbm)
    plsc.subcore_barrier()
    dump(e1_hbm)
    zero_acc()
    plsc.subcore_barrier()
    spmm(e1_hbm)
    plsc.subcore_barrier()
    dump(e2_hbm)
    plsc.subcore_barrier()

    b0 = sid * BSUB

    def gather_sum(ids_hbm, dst):
        pltpu.sync_copy(ids_hbm.at[pl.ds(b0, BSUB)], idxb)
        pltpu.async_copy(x_hbm.at[h].at[idxb], dst, sem_g0)
        pltpu.async_copy(e1_hbm.at[h].at[idxb], gath0.at[pl.ds(0, BSUB)], sem_g0)
        pltpu.async_copy(e2_hbm.at[h].at[idxb], gath1.at[pl.ds(0, BSUB)], sem_g0)
        pltpu.make_async_copy(x_hbm.at[h].at[idxb], dst, sem_g0).wait()
        pltpu.make_async_copy(e1_hbm.at[h].at[idxb], gath0.at[pl.ds(0, BSUB)],
                              sem_g0).wait()
        pltpu.make_async_copy(e2_hbm.at[h].at[idxb], gath1.at[pl.ds(0, BSUB)],
                              sem_g0).wait()

        @plsc.parallel_loop(0, BSUB, unroll=4)
        def _acc(r):
            dst[r] = dst[r] + gath0[r] + gath1[r]

    def dots(out_buf):
        @plsc.parallel_loop(0, BSUB // 16)
        def _dot(gi):
            s_acc = jnp.zeros((16,), jnp.float32)
            for k in range(16):
                bb = gi * 16 + k
                s = _hsum(sum_a[bb] * sum_b[bb])
                s_acc = jnp.where(lanes == k, s, s_acc)
            out_buf[pl.ds(gi * 16, 16)] = s_acc

    gather_sum(user_hbm, sum_a)
    gather_sum(posi_hbm, sum_b)
    dots(pos_buf)
    gather_sum(negi_hbm, sum_b)
    dots(neg_buf)

    pltpu.sync_copy(pos_buf, pos_out.at[h].at[pl.ds(b0, BSUB)])
    pltpu.sync_copy(neg_buf, neg_out.at[h].at[pl.ds(b0, BSUB)])


def kernel(user_emb_w, item_emb_w, snm_vals, snm_rows, snm_cols, user, pos_item, neg_item):
    e0 = jnp.concatenate([user_emb_w, item_emb_w,
                          jnp.zeros((N_PAD - N_NODES, D), jnp.float32)], axis=0)
    x2 = jnp.stack([e0[:, :DH], e0[:, DH:]])
    pos_p, neg_p, _e1, _e2 = _lightgcn(
        x2, snm_rows, snm_cols, snm_vals,
        user, pos_item + HALF_N, neg_item + HALF_N)
    pos = ((pos_p[0] + pos_p[1]) * (1.0 / 9.0)).reshape(BATCH, 1)
    neg = ((neg_p[0] + neg_p[1]) * (1.0 / 9.0)).reshape(BATCH, 1)
    return (pos, neg)
